# baseline (speedup 1.0000x reference)
import functools

import jax
import jax.numpy as jnp
from jax import lax
from jax.experimental import pallas as pl
from jax.experimental.pallas import tpu as pltpu
from jax.experimental.pallas import tpu_sc as plsc

_N = 10000
_E = 160000
_D_IN = 2020
_H = 20

_NTILES = 32
_EB = 128
_EPAD = 163840
_EPT = _EPAD // _NTILES
_NBLK = _EPT // _EB
_ACC_ROWS = 10240
_DQ = 128
_DD = 128
_ZR = 16
_RPT = _ACC_ROWS // 16
_OPT = 624

_mesh = plsc.VectorSubcoreMesh(core_axis_name="c", subcore_axis_name="s")


@functools.partial(
    pl.kernel,
    out_type=jax.ShapeDtypeStruct((2, _N, _DQ), jnp.float32),
    mesh=_mesh,
    scratch_types=[
        pltpu.VMEM((_NBLK, _EB), jnp.int32),
        pltpu.VMEM((_NBLK, _EB), jnp.int32),
        pltpu.VMEM((2, _EB, _DQ), jnp.float32),
        pltpu.VMEM((_ZR, _DQ), jnp.float32),
        pltpu.VMEM_SHARED((_ACC_ROWS, _DQ), jnp.float32),
        pltpu.SemaphoreType.DMA,
        pltpu.SemaphoreType.DMA,
    ],
)
def _q_sc(g_hbm, src_hbm, dst_hbm, out_hbm, src_all, dst_all, rows, zbuf,
          acc, s0, s1):
    c = lax.axis_index("c")
    s = lax.axis_index("s")
    sems = [s0, s1]
    r0 = s * _OPT
    wid = c * 16 + s
    pltpu.sync_copy(src_hbm.at[pl.ds(wid * _NBLK, _NBLK)], src_all)
    pltpu.sync_copy(dst_hbm.at[pl.ds(wid * _NBLK, _NBLK)], dst_all)

    zv = jnp.zeros((16,), jnp.float32)

    def zrow(r, carry):
        for j in range(_DQ // 16):
            zbuf[r, pl.ds(16 * j, 16)] = zv
        return carry

    lax.fori_loop(0, _ZR, zrow, 0)

    def zcp(i, carry):
        pltpu.sync_copy(zbuf, acc.at[pl.ds(s * _RPT + i * _ZR, _ZR)])
        return carry

    lax.fori_loop(0, _RPT // _ZR, zcp, 0)
    plsc.subcore_barrier()

    for b in range(2):
        pltpu.make_async_copy(g_hbm.at[src_all.at[b]], rows.at[b],
                              sems[b]).start()

    def body(i, carry):
        j0 = i * 2
        for b in range(2):
            j = j0 + b
            pltpu.make_async_copy(g_hbm.at[src_all.at[j]], rows.at[b],
                                  sems[b]).wait()
            pltpu.sync_copy(rows.at[b], acc.at[dst_all.at[j]], add=True)
            nj = j + 2

            @pl.when(nj < _NBLK)
            def _():
                pltpu.make_async_copy(g_hbm.at[src_all.at[nj]], rows.at[b],
                                      sems[b]).start()
        return carry

    lax.fori_loop(0, _NBLK // 2, body, 0)
    plsc.subcore_barrier()

    @pl.when(s < 15)
    def _():
        pltpu.sync_copy(acc.at[pl.ds(r0, _OPT)],
                        out_hbm.at[c].at[pl.ds(r0, _OPT)])

    @pl.when(s == 15)
    def _():
        pltpu.sync_copy(acc.at[pl.ds(15 * _OPT, _N - 15 * _OPT)],
                        out_hbm.at[c].at[pl.ds(15 * _OPT, _N - 15 * _OPT)])


@functools.partial(
    pl.kernel,
    out_type=jax.ShapeDtypeStruct((2, _N, _DD), jnp.float32),
    mesh=_mesh,
    scratch_types=[
        pltpu.VMEM((_NBLK, _EB), jnp.int32),
        pltpu.VMEM((_EB, _DD), jnp.float32),
        pltpu.VMEM((_ZR, _DD), jnp.float32),
        pltpu.VMEM_SHARED((_ACC_ROWS, _DD), jnp.float32),
    ],
)
def _deg_sc(src_hbm, out_hbm, src_all, ones_v, zbuf, acc):
    c = lax.axis_index("c")
    s = lax.axis_index("s")
    wid = c * 16 + s
    pltpu.sync_copy(src_hbm.at[pl.ds(wid * _NBLK, _NBLK)], src_all)
    zv = jnp.zeros((16,), jnp.float32)
    ov = jnp.ones((16,), jnp.float32)

    def fill(r, carry):
        for j in range(_DD // 16):
            zbuf[r, pl.ds(16 * j, 16)] = zv
        return carry

    lax.fori_loop(0, _ZR, fill, 0)

    def fill1(r, carry):
        for j in range(_DD // 16):
            ones_v[r, pl.ds(16 * j, 16)] = ov
        return carry

    lax.fori_loop(0, _EB, fill1, 0)

    def zcp(i, carry):
        pltpu.sync_copy(zbuf, acc.at[pl.ds(s * _RPT + i * _ZR, _ZR)])
        return carry

    lax.fori_loop(0, _RPT // _ZR, zcp, 0)
    plsc.subcore_barrier()

    def body(it, carry):
        pltpu.sync_copy(ones_v, acc.at[src_all.at[it]], add=True)
        return carry

    lax.fori_loop(0, _NBLK, body, 0)
    plsc.subcore_barrier()
    r0 = s * _OPT

    @pl.when(s < 15)
    def _():
        pltpu.sync_copy(acc.at[pl.ds(r0, _OPT)],
                        out_hbm.at[c].at[pl.ds(r0, _OPT)])

    @pl.when(s == 15)
    def _():
        pltpu.sync_copy(acc.at[pl.ds(15 * _OPT, _N - 15 * _OPT)],
                        out_hbm.at[c].at[pl.ds(15 * _OPT, _N - 15 * _OPT)])


_RB = 1000


def _tca_body(feat_ref, w0c_ref, w1p_ref, w2p_ref, degp_ref,
              u0_ref, u1_ref, g1_ref, dis_ref):
    x = feat_ref[...]
    deg = degp_ref[...][0, :, 0] + degp_ref[...][1, :, 0]
    dis = jnp.where(deg > 0, lax.rsqrt(jnp.maximum(deg, 1e-12)), 0.0)
    discol = dis[:, None]
    u0_ref[...] = jnp.dot(x, w0c_ref[...], preferred_element_type=jnp.float32)
    u1_ref[...] = jnp.dot(x, w1p_ref[...], preferred_element_type=jnp.float32)
    g1_ref[...] = discol * jnp.dot(x, w2p_ref[...],
                                   preferred_element_type=jnp.float32)
    dis_ref[...] = discol


def _tcb_body(s1p_ref, dis_ref, u1_ref, g2_ref):
    dis = dis_ref[...]
    s1 = s1p_ref[...][0] + s1p_ref[...][1]
    g2_ref[...] = dis * u1_ref[...] - 2.0 * (dis * dis) * s1


def _tcc_body(has_resid, lw_idx,
              s2p_ref, dis_ref, u0_ref, b_ref, bg_ref, bb_ref, xprev_ref,
              eacc_ref, lw_ref, w0c_ref, w1p_ref, w2p_ref,
              x_ref, eout_ref, u0n_ref, u1n_ref, g1n_ref):
    dis = dis_ref[...]
    s2 = s2p_ref[...][0] + s2p_ref[...][1]
    cheb = u0_ref[...] - dis * s2[:, 0:_H] + b_ref[...][0]
    scale = 1.0 / jnp.sqrt(jnp.float32(1.0 + 1e-05))
    v = jax.nn.relu(cheb * scale * bg_ref[...][0] + bb_ref[...][0])
    if has_resid:
        v = v + 0.7 * xprev_ref[...]
    x_ref[...] = v
    w = jax.nn.softmax(lw_ref[...][0])
    e = w[lw_idx] * v
    if has_resid:
        e = e + eacc_ref[...]
    eout_ref[...] = e
    u0n_ref[...] = jnp.dot(v, w0c_ref[...], preferred_element_type=jnp.float32)
    u1n_ref[...] = jnp.dot(v, w1p_ref[...], preferred_element_type=jnp.float32)
    g1n_ref[...] = dis * jnp.dot(v, w2p_ref[...],
                                 preferred_element_type=jnp.float32)


def _tcf_body(s2p_ref, dis_ref, u0_ref, b_ref, bg_ref, bb_ref, xprev_ref,
              eacc_ref, lw_ref, ow_ref, ob_ref, out_ref):
    dis = dis_ref[...]
    s2 = s2p_ref[...][0] + s2p_ref[...][1]
    cheb = u0_ref[...] - dis * s2[:, 0:_H] + b_ref[...][0]
    scale = 1.0 / jnp.sqrt(jnp.float32(1.0 + 1e-05))
    v = jax.nn.relu(cheb * scale * bg_ref[...][0] + bb_ref[...][0]) \
        + 0.7 * xprev_ref[...]
    w = jax.nn.softmax(lw_ref[...][0])
    emb = eacc_ref[...] + w[3] * v
    out_ref[...] = jnp.dot(emb, ow_ref[...].T,
                           preferred_element_type=jnp.float32) + ob_ref[...][0]


def _full(shape):
    return pl.BlockSpec(shape, lambda *_: tuple(0 for _ in shape))


_tca = pl.pallas_call(
    _tca_body,
    grid=(_N // _RB,),
    in_specs=[
        pl.BlockSpec((_RB, _D_IN), lambda i: (i, 0)),
        pl.BlockSpec((_D_IN, _H), lambda i: (0, 0)),
        pl.BlockSpec((_D_IN, _DQ), lambda i: (0, 0)),
        pl.BlockSpec((_D_IN, _DQ), lambda i: (0, 0)),
        pl.BlockSpec((2, _RB, _DD), lambda i: (0, i, 0)),
    ],
    out_specs=[
        pl.BlockSpec((_RB, _H), lambda i: (i, 0)),
        pl.BlockSpec((_RB, _DQ), lambda i: (i, 0)),
        pl.BlockSpec((_RB, _DQ), lambda i: (i, 0)),
        pl.BlockSpec((_RB, 1), lambda i: (i, 0)),
    ],
    out_shape=[
        jax.ShapeDtypeStruct((_N, _H), jnp.float32),
        jax.ShapeDtypeStruct((_N, _DQ), jnp.float32),
        jax.ShapeDtypeStruct((_N, _DQ), jnp.float32),
        jax.ShapeDtypeStruct((_N, 1), jnp.float32),
    ],
)

def _rows(w):
    return pl.BlockSpec((_RB, w), lambda i: (i, 0))


def _part(w):
    return pl.BlockSpec((2, _RB, w), lambda i: (0, i, 0))


_tcb = pl.pallas_call(
    _tcb_body,
    grid=(_N // _RB,),
    in_specs=[_part(_DQ), _rows(1), _rows(_DQ)],
    out_specs=_rows(_DQ),
    out_shape=jax.ShapeDtypeStruct((_N, _DQ), jnp.float32),
)


def _make_tcc(has_resid, lw_idx):
    return pl.pallas_call(
        functools.partial(_tcc_body, has_resid, lw_idx),
        grid=(_N // _RB,),
        in_specs=[
            _part(_DQ), _rows(1), _rows(_H),
            _full((1, _H)), _full((1, _H)), _full((1, _H)),
            _rows(_H), _rows(_H), _full((1, 4)),
            _full((_H, _H)), _full((_H, _DQ)), _full((_H, _DQ)),
        ],
        out_specs=[
            _rows(_H), _rows(_H), _rows(_H), _rows(_DQ), _rows(_DQ),
        ],
        out_shape=[
            jax.ShapeDtypeStruct((_N, _H), jnp.float32),
            jax.ShapeDtypeStruct((_N, _H), jnp.float32),
            jax.ShapeDtypeStruct((_N, _H), jnp.float32),
            jax.ShapeDtypeStruct((_N, _DQ), jnp.float32),
            jax.ShapeDtypeStruct((_N, _DQ), jnp.float32),
        ],
    )


_tcf = pl.pallas_call(
    _tcf_body,
    grid=(_N // _RB,),
    in_specs=[
        _part(_DQ), _rows(1), _rows(_H),
        _full((1, _H)), _full((1, _H)), _full((1, _H)),
        _rows(_H), _rows(_H), _full((1, 4)),
        _full((2, _H)), _full((1, 2)),
    ],
    out_specs=_rows(2),
    out_shape=jax.ShapeDtypeStruct((_N, 2), jnp.float32),
)


def _pad_w(w):
    return jnp.concatenate(
        [w, jnp.zeros((w.shape[0], _DQ - w.shape[1]), jnp.float32)], axis=1)


def kernel(features, edges, edge_weight, c0w0, c0w1, c0w2, c0b, c1w0, c1w1,
           c1w2, c1b, c2w0, c2w1, c2w2, c2b, c3w0, c3w1, c3w2, c3b, bn0g,
           bn0b, bn1g, bn1b, bn2g, bn2b, bn3g, bn3b, out_w, out_b, layer_w):
    src = edges[0]
    dst = edges[1]
    npad = _EPAD - _E
    nrow = _EPAD // _EB
    src_q = jnp.concatenate([src, jnp.zeros((npad,), jnp.int32)])
    src_q = src_q.reshape(nrow, _EB)
    dst_q = jnp.concatenate([dst, jnp.full((npad,), _N, jnp.int32)])
    dst_q = dst_q.reshape(nrow, _EB)
    src_d = jnp.concatenate([src, jnp.full((npad,), _N, jnp.int32)])
    src_d = src_d.reshape(nrow, _EB)

    ws = [(c0w0, c0w1, c0w2), (c1w0, c1w1, c1w2), (c2w0, c2w1, c2w2),
          (c3w0, c3w1, c3w2)]
    w0c = [w0 - w2 for (w0, w1, w2) in ws]
    w1p = [_pad_w(w1) for (w0, w1, w2) in ws]
    w2p = [_pad_w(w2) for (w0, w1, w2) in ws]
    bs = [c0b, c1b, c2b, c3b]
    bgs = [bn0g, bn1g, bn2g, bn3g]
    bbs = [bn0b, bn1b, bn2b, bn3b]

    def row(v):
        return v.reshape(1, -1)

    degp = _deg_sc(src_d)
    u0, u1, g1, dis = _tca(features, w0c[0], w1p[0], w2p[0], degp)

    x = jnp.zeros((_N, _H), jnp.float32)
    eacc = jnp.zeros((_N, _H), jnp.float32)
    for i in range(4):
        s1p = _q_sc(g1, src_q, dst_q)
        g2 = _tcb(s1p, dis, u1)
        s2p = _q_sc(g2, src_q, dst_q)
        if i < 3:
            x, eacc, u0, u1, g1 = _make_tcc(i > 0, i)(
                s2p, dis, u0, row(bs[i]), row(bgs[i]), row(bbs[i]), x, eacc,
                row(layer_w), w0c[i + 1], w1p[i + 1], w2p[i + 1])
        else:
            out = _tcf(s2p, dis, u0, row(bs[i]), row(bgs[i]), row(bbs[i]), x,
                       eacc, row(layer_w), out_w, row(out_b))
    return out


# speedup vs baseline: 1.1566x; 1.1566x over previous
import functools

import jax
import jax.numpy as jnp
from jax import lax
from jax.experimental import pallas as pl
from jax.experimental.pallas import tpu as pltpu
from jax.experimental.pallas import tpu_sc as plsc

_N = 10000
_E = 160000
_D_IN = 2020
_H = 20

_NTILES = 32
_EB = 128
_EPAD = 163840
_EPT = _EPAD // _NTILES
_NBLK = _EPT // _EB
_ACC_ROWS = 10240
_DQ = 128
_DD = 128
_DW = 32
_ZR = 16
_RPT = _ACC_ROWS // 16
_OPT = 624

_mesh = plsc.VectorSubcoreMesh(core_axis_name="c", subcore_axis_name="s")


@functools.partial(
    pl.kernel,
    out_type=jax.ShapeDtypeStruct((2, _N, _DQ), jnp.float32),
    mesh=_mesh,
    scratch_types=[
        pltpu.VMEM((_NBLK, _EB), jnp.int32),
        pltpu.VMEM((_NBLK, _EB), jnp.int32),
        pltpu.VMEM((2, _EB, _DQ), jnp.float32),
        pltpu.VMEM_SHARED((_ACC_ROWS, _DQ), jnp.float32),
        pltpu.SemaphoreType.DMA,
        pltpu.SemaphoreType.DMA,
    ],
)
def _q_sc(g_hbm, src_hbm, dst_hbm, zeros_hbm, out_hbm, src_all, dst_all,
          rows, acc, s0, s1):
    c = lax.axis_index("c")
    s = lax.axis_index("s")
    sems = [s0, s1]
    r0 = s * _OPT
    wid = c * 16 + s
    pltpu.sync_copy(src_hbm.at[pl.ds(wid * _NBLK, _NBLK)], src_all)
    pltpu.sync_copy(dst_hbm.at[pl.ds(wid * _NBLK, _NBLK)], dst_all)
    pltpu.sync_copy(zeros_hbm, acc.at[pl.ds(s * _RPT, _RPT)])
    plsc.subcore_barrier()

    for b in range(2):
        pltpu.make_async_copy(g_hbm.at[src_all.at[b]], rows.at[b],
                              sems[b]).start()

    def body(i, carry):
        j0 = i * 2
        for b in range(2):
            j = j0 + b
            pltpu.make_async_copy(g_hbm.at[src_all.at[j]], rows.at[b],
                                  sems[b]).wait()
            pltpu.sync_copy(rows.at[b], acc.at[dst_all.at[j]], add=True)
            nj = j + 2

            @pl.when(nj < _NBLK)
            def _():
                pltpu.make_async_copy(g_hbm.at[src_all.at[nj]], rows.at[b],
                                      sems[b]).start()
        return carry

    lax.fori_loop(0, _NBLK // 2, body, 0)
    plsc.subcore_barrier()

    @pl.when(s < 15)
    def _():
        pltpu.sync_copy(acc.at[pl.ds(r0, _OPT)],
                        out_hbm.at[c].at[pl.ds(r0, _OPT)])

    @pl.when(s == 15)
    def _():
        pltpu.sync_copy(acc.at[pl.ds(15 * _OPT, _N - 15 * _OPT)],
                        out_hbm.at[c].at[pl.ds(15 * _OPT, _N - 15 * _OPT)])


@functools.partial(
    pl.kernel,
    out_type=jax.ShapeDtypeStruct((2, _N, _DD), jnp.float32),
    mesh=_mesh,
    scratch_types=[
        pltpu.VMEM((_NBLK, _EB), jnp.int32),
        pltpu.VMEM((_EB, _DD), jnp.float32),
        pltpu.VMEM_SHARED((_ACC_ROWS, _DD), jnp.float32),
    ],
)
def _deg_sc(src_hbm, zeros_hbm, ones_hbm, out_hbm, src_all, ones_v, acc):
    c = lax.axis_index("c")
    s = lax.axis_index("s")
    wid = c * 16 + s
    pltpu.sync_copy(src_hbm.at[pl.ds(wid * _NBLK, _NBLK)], src_all)
    pltpu.sync_copy(ones_hbm, ones_v)
    pltpu.sync_copy(zeros_hbm, acc.at[pl.ds(s * _RPT, _RPT)])
    plsc.subcore_barrier()

    def body(it, carry):
        pltpu.sync_copy(ones_v, acc.at[src_all.at[it]], add=True)
        return carry

    lax.fori_loop(0, _NBLK, body, 0)
    plsc.subcore_barrier()
    r0 = s * _OPT

    @pl.when(s < 15)
    def _():
        pltpu.sync_copy(acc.at[pl.ds(r0, _OPT)],
                        out_hbm.at[c].at[pl.ds(r0, _OPT)])

    @pl.when(s == 15)
    def _():
        pltpu.sync_copy(acc.at[pl.ds(15 * _OPT, _N - 15 * _OPT)],
                        out_hbm.at[c].at[pl.ds(15 * _OPT, _N - 15 * _OPT)])


_RB = 1000


def _tca_body(feat_ref, w0c_ref, w1p_ref, w2p_ref, degp_ref,
              u0_ref, u1_ref, g1_ref, dis_ref):
    x = feat_ref[...]
    deg = degp_ref[...][0, :, 0] + degp_ref[...][1, :, 0]
    dis = jnp.where(deg > 0, lax.rsqrt(jnp.maximum(deg, 1e-12)), 0.0)
    discol = dis[:, None]
    u0_ref[...] = jnp.dot(x, w0c_ref[...], preferred_element_type=jnp.float32)
    u1_ref[...] = jnp.dot(x, w1p_ref[...], preferred_element_type=jnp.float32)
    g1_ref[...] = discol * jnp.dot(x, w2p_ref[...],
                                   preferred_element_type=jnp.float32)
    dis_ref[...] = discol


def _tcb_body(s1p_ref, dis_ref, u1_ref, g2_ref):
    dis = dis_ref[...]
    s1 = s1p_ref[...][0] + s1p_ref[...][1]
    g2_ref[...] = dis * u1_ref[...] - 2.0 * (dis * dis) * s1


def _tcc_body(has_resid, lw_idx,
              s2p_ref, dis_ref, u0_ref, b_ref, bg_ref, bb_ref, xprev_ref,
              eacc_ref, lw_ref, w0c_ref, w1p_ref, w2p_ref,
              x_ref, eout_ref, u0n_ref, u1n_ref, g1n_ref):
    dis = dis_ref[...]
    s2 = s2p_ref[...][0] + s2p_ref[...][1]
    cheb = u0_ref[...] - dis * s2[:, 0:_H] + b_ref[...][0]
    scale = 1.0 / jnp.sqrt(jnp.float32(1.0 + 1e-05))
    v = jax.nn.relu(cheb * scale * bg_ref[...][0] + bb_ref[...][0])
    if has_resid:
        v = v + 0.7 * xprev_ref[...]
    x_ref[...] = v
    w = jax.nn.softmax(lw_ref[...][0])
    e = w[lw_idx] * v
    if has_resid:
        e = e + eacc_ref[...]
    eout_ref[...] = e
    u0n_ref[...] = jnp.dot(v, w0c_ref[...], preferred_element_type=jnp.float32)
    u1n_ref[...] = jnp.dot(v, w1p_ref[...], preferred_element_type=jnp.float32)
    g1n_ref[...] = dis * jnp.dot(v, w2p_ref[...],
                                 preferred_element_type=jnp.float32)


def _tcf_body(s2p_ref, dis_ref, u0_ref, b_ref, bg_ref, bb_ref, xprev_ref,
              eacc_ref, lw_ref, ow_ref, ob_ref, out_ref):
    dis = dis_ref[...]
    s2 = s2p_ref[...][0] + s2p_ref[...][1]
    cheb = u0_ref[...] - dis * s2[:, 0:_H] + b_ref[...][0]
    scale = 1.0 / jnp.sqrt(jnp.float32(1.0 + 1e-05))
    v = jax.nn.relu(cheb * scale * bg_ref[...][0] + bb_ref[...][0]) \
        + 0.7 * xprev_ref[...]
    w = jax.nn.softmax(lw_ref[...][0])
    emb = eacc_ref[...] + w[3] * v
    out_ref[...] = jnp.dot(emb, ow_ref[...].T,
                           preferred_element_type=jnp.float32) + ob_ref[...][0]


def _full(shape):
    return pl.BlockSpec(shape, lambda *_: tuple(0 for _ in shape))


_tca = pl.pallas_call(
    _tca_body,
    grid=(_N // _RB,),
    in_specs=[
        pl.BlockSpec((_RB, _D_IN), lambda i: (i, 0)),
        pl.BlockSpec((_D_IN, _H), lambda i: (0, 0)),
        pl.BlockSpec((_D_IN, _DQ), lambda i: (0, 0)),
        pl.BlockSpec((_D_IN, _DQ), lambda i: (0, 0)),
        pl.BlockSpec((2, _RB, _DD), lambda i: (0, i, 0)),
    ],
    out_specs=[
        pl.BlockSpec((_RB, _H), lambda i: (i, 0)),
        pl.BlockSpec((_RB, _DQ), lambda i: (i, 0)),
        pl.BlockSpec((_RB, _DQ), lambda i: (i, 0)),
        pl.BlockSpec((_RB, 1), lambda i: (i, 0)),
    ],
    out_shape=[
        jax.ShapeDtypeStruct((_N, _H), jnp.float32),
        jax.ShapeDtypeStruct((_N, _DQ), jnp.float32),
        jax.ShapeDtypeStruct((_N, _DQ), jnp.float32),
        jax.ShapeDtypeStruct((_N, 1), jnp.float32),
    ],
)

def _rows(w):
    return pl.BlockSpec((_RB, w), lambda i: (i, 0))


def _part(w):
    return pl.BlockSpec((2, _RB, w), lambda i: (0, i, 0))


_tcb = pl.pallas_call(
    _tcb_body,
    grid=(_N // _RB,),
    in_specs=[_part(_DQ), _rows(1), _rows(_DQ)],
    out_specs=_rows(_DQ),
    out_shape=jax.ShapeDtypeStruct((_N, _DQ), jnp.float32),
)


def _make_tcc(has_resid, lw_idx):
    return pl.pallas_call(
        functools.partial(_tcc_body, has_resid, lw_idx),
        grid=(_N // _RB,),
        in_specs=[
            _part(_DQ), _rows(1), _rows(_H),
            _full((1, _H)), _full((1, _H)), _full((1, _H)),
            _rows(_H), _rows(_H), _full((1, 4)),
            _full((_H, _H)), _full((_H, _DQ)), _full((_H, _DQ)),
        ],
        out_specs=[
            _rows(_H), _rows(_H), _rows(_H), _rows(_DQ), _rows(_DQ),
        ],
        out_shape=[
            jax.ShapeDtypeStruct((_N, _H), jnp.float32),
            jax.ShapeDtypeStruct((_N, _H), jnp.float32),
            jax.ShapeDtypeStruct((_N, _H), jnp.float32),
            jax.ShapeDtypeStruct((_N, _DQ), jnp.float32),
            jax.ShapeDtypeStruct((_N, _DQ), jnp.float32),
        ],
    )


_tcf = pl.pallas_call(
    _tcf_body,
    grid=(_N // _RB,),
    in_specs=[
        _part(_DQ), _rows(1), _rows(_H),
        _full((1, _H)), _full((1, _H)), _full((1, _H)),
        _rows(_H), _rows(_H), _full((1, 4)),
        _full((2, _H)), _full((1, 2)),
    ],
    out_specs=_rows(2),
    out_shape=jax.ShapeDtypeStruct((_N, 2), jnp.float32),
)


def _pad_w(w):
    return jnp.concatenate(
        [w, jnp.zeros((w.shape[0], _DQ - w.shape[1]), jnp.float32)], axis=1)


def kernel(features, edges, edge_weight, c0w0, c0w1, c0w2, c0b, c1w0, c1w1,
           c1w2, c1b, c2w0, c2w1, c2w2, c2b, c3w0, c3w1, c3w2, c3b, bn0g,
           bn0b, bn1g, bn1b, bn2g, bn2b, bn3g, bn3b, out_w, out_b, layer_w):
    src = edges[0]
    dst = edges[1]
    npad = _EPAD - _E
    nrow = _EPAD // _EB
    src_q = jnp.concatenate([src, jnp.zeros((npad,), jnp.int32)])
    src_q = src_q.reshape(nrow, _EB)
    dst_q = jnp.concatenate([dst, jnp.full((npad,), _N, jnp.int32)])
    dst_q = dst_q.reshape(nrow, _EB)
    src_d = jnp.concatenate([src, jnp.full((npad,), _N, jnp.int32)])
    src_d = src_d.reshape(nrow, _EB)

    ws = [(c0w0, c0w1, c0w2), (c1w0, c1w1, c1w2), (c2w0, c2w1, c2w2),
          (c3w0, c3w1, c3w2)]
    w0c = [w0 - w2 for (w0, w1, w2) in ws]
    w1p = [_pad_w(w1) for (w0, w1, w2) in ws]
    w2p = [_pad_w(w2) for (w0, w1, w2) in ws]
    bs = [c0b, c1b, c2b, c3b]
    bgs = [bn0g, bn1g, bn2g, bn3g]
    bbs = [bn0b, bn1b, bn2b, bn3b]

    def row(v):
        return v.reshape(1, -1)

    zeros_ = jnp.zeros((_RPT, _DQ), jnp.float32)
    ones_ = jnp.ones((_EB, _DD), jnp.float32)

    degp = _deg_sc(src_d, zeros_, ones_)
    u0, u1, g1, dis = _tca(features, w0c[0], w1p[0], w2p[0], degp)

    x = jnp.zeros((_N, _H), jnp.float32)
    eacc = jnp.zeros((_N, _H), jnp.float32)
    for i in range(4):
        s1p = _q_sc(g1, src_q, dst_q, zeros_)
        g2 = _tcb(s1p, dis, u1)
        s2p = _q_sc(g2, src_q, dst_q, zeros_)
        if i < 3:
            x, eacc, u0, u1, g1 = _make_tcc(i > 0, i)(
                s2p, dis, u0, row(bs[i]), row(bgs[i]), row(bbs[i]), x, eacc,
                row(layer_w), w0c[i + 1], w1p[i + 1], w2p[i + 1])
        else:
            out = _tcf(s2p, dis, u0, row(bs[i]), row(bgs[i]), row(bbs[i]), x,
                       eacc, row(layer_w), out_w, row(out_b))
    return out


# speedup vs baseline: 1.1608x; 1.0037x over previous
import functools

import jax
import jax.numpy as jnp
from jax import lax
from jax.experimental import pallas as pl
from jax.experimental.pallas import tpu as pltpu
from jax.experimental.pallas import tpu_sc as plsc

_N = 10000
_E = 160000
_D_IN = 2020
_H = 20

_NTILES = 32
_EB = 128
_EPAD = 163840
_EPT = _EPAD // _NTILES
_NBLK = _EPT // _EB
_ACC_ROWS = 10240
_DQ = 128
_DD = 128
_RPT = _ACC_ROWS // 16
_OPT = 624

_mesh = plsc.VectorSubcoreMesh(core_axis_name="c", subcore_axis_name="s")


@functools.partial(
    pl.kernel,
    out_type=jax.ShapeDtypeStruct((2, _N, _DQ), jnp.float32),
    mesh=_mesh,
    scratch_types=[
        pltpu.VMEM((_NBLK, _EB), jnp.int32),
        pltpu.VMEM((_NBLK, _EB), jnp.int32),
        pltpu.VMEM((2, _EB, _DQ), jnp.float32),
        pltpu.VMEM_SHARED((_ACC_ROWS, _DQ), jnp.float32),
        pltpu.SemaphoreType.DMA,
        pltpu.SemaphoreType.DMA,
    ],
)
def _q_sc(g_hbm, src_hbm, dst_hbm, zeros_hbm, out_hbm, src_all, dst_all,
          rows, acc, s0, s1):
    c = lax.axis_index("c")
    s = lax.axis_index("s")
    sems = [s0, s1]
    r0 = s * _OPT
    wid = c * 16 + s
    pltpu.sync_copy(src_hbm.at[pl.ds(wid * _NBLK, _NBLK)], src_all)
    pltpu.sync_copy(dst_hbm.at[pl.ds(wid * _NBLK, _NBLK)], dst_all)
    pltpu.sync_copy(zeros_hbm, acc.at[pl.ds(s * _RPT, _RPT)])
    plsc.subcore_barrier()

    for b in range(2):
        pltpu.make_async_copy(g_hbm.at[src_all.at[b]], rows.at[b],
                              sems[b]).start()

    def body(i, carry):
        j0 = i * 2
        for b in range(2):
            j = j0 + b
            pltpu.make_async_copy(g_hbm.at[src_all.at[j]], rows.at[b],
                                  sems[b]).wait()
            pltpu.sync_copy(rows.at[b], acc.at[dst_all.at[j]], add=True)
            nj = j + 2

            @pl.when(nj < _NBLK)
            def _():
                pltpu.make_async_copy(g_hbm.at[src_all.at[nj]], rows.at[b],
                                      sems[b]).start()
        return carry

    lax.fori_loop(0, _NBLK // 2, body, 0)
    plsc.subcore_barrier()

    @pl.when(s < 15)
    def _():
        pltpu.sync_copy(acc.at[pl.ds(r0, _OPT)],
                        out_hbm.at[c].at[pl.ds(r0, _OPT)])

    @pl.when(s == 15)
    def _():
        pltpu.sync_copy(acc.at[pl.ds(15 * _OPT, _N - 15 * _OPT)],
                        out_hbm.at[c].at[pl.ds(15 * _OPT, _N - 15 * _OPT)])


@functools.partial(
    pl.kernel,
    out_type=jax.ShapeDtypeStruct((2, _N, _DD), jnp.float32),
    mesh=_mesh,
    scratch_types=[
        pltpu.VMEM((_NBLK, _EB), jnp.int32),
        pltpu.VMEM((_EB, _DD), jnp.float32),
        pltpu.VMEM_SHARED((_ACC_ROWS, _DD), jnp.float32),
    ],
)
def _deg_sc(src_hbm, zeros_hbm, ones_hbm, out_hbm, src_all, ones_v, acc):
    c = lax.axis_index("c")
    s = lax.axis_index("s")
    wid = c * 16 + s
    pltpu.sync_copy(src_hbm.at[pl.ds(wid * _NBLK, _NBLK)], src_all)
    pltpu.sync_copy(ones_hbm, ones_v)
    pltpu.sync_copy(zeros_hbm, acc.at[pl.ds(s * _RPT, _RPT)])
    plsc.subcore_barrier()

    def body(it, carry):
        pltpu.sync_copy(ones_v, acc.at[src_all.at[it]], add=True)
        return carry

    lax.fori_loop(0, _NBLK, body, 0)
    plsc.subcore_barrier()
    r0 = s * _OPT

    @pl.when(s < 15)
    def _():
        pltpu.sync_copy(acc.at[pl.ds(r0, _OPT)],
                        out_hbm.at[c].at[pl.ds(r0, _OPT)])

    @pl.when(s == 15)
    def _():
        pltpu.sync_copy(acc.at[pl.ds(15 * _OPT, _N - 15 * _OPT)],
                        out_hbm.at[c].at[pl.ds(15 * _OPT, _N - 15 * _OPT)])


_RB = 1000


def _tca_body(feat_ref, w0c_ref, w1p_ref, w2p_ref, degp_ref,
              u0_ref, u1_ref, g1_ref, dis_ref):
    x = feat_ref[...]
    deg = degp_ref[...][0, :, 0] + degp_ref[...][1, :, 0]
    dis = jnp.where(deg > 0, lax.rsqrt(jnp.maximum(deg, 1e-12)), 0.0)
    discol = dis[:, None]
    u0_ref[...] = jnp.dot(x, w0c_ref[...], preferred_element_type=jnp.float32)
    u1_ref[...] = jnp.dot(x, w1p_ref[...], preferred_element_type=jnp.float32)
    g1_ref[...] = discol * jnp.dot(x, w2p_ref[...],
                                   preferred_element_type=jnp.float32)
    dis_ref[...] = discol


def _tcb_body(s1p_ref, dis_ref, u1_ref, g2_ref):
    dis = dis_ref[...]
    s1 = s1p_ref[...][0] + s1p_ref[...][1]
    g2_ref[...] = dis * u1_ref[...] - 2.0 * (dis * dis) * s1


def _tcc_body(has_resid, lw_idx,
              s2p_ref, dis_ref, u0_ref, b_ref, bg_ref, bb_ref, xprev_ref,
              eacc_ref, lw_ref, w0c_ref, w1p_ref, w2p_ref,
              x_ref, eout_ref, u0n_ref, u1n_ref, g1n_ref):
    dis = dis_ref[...]
    s2 = s2p_ref[...][0] + s2p_ref[...][1]
    cheb = u0_ref[...] - dis * s2[:, 0:_H] + b_ref[...][0]
    scale = 1.0 / jnp.sqrt(jnp.float32(1.0 + 1e-05))
    v = jax.nn.relu(cheb * scale * bg_ref[...][0] + bb_ref[...][0])
    if has_resid:
        v = v + 0.7 * xprev_ref[...]
    x_ref[...] = v
    w = jax.nn.softmax(lw_ref[...][0])
    e = w[lw_idx] * v
    if has_resid:
        e = e + eacc_ref[...]
    eout_ref[...] = e
    u0n_ref[...] = jnp.dot(v, w0c_ref[...], preferred_element_type=jnp.float32)
    u1n_ref[...] = jnp.dot(v, w1p_ref[...], preferred_element_type=jnp.float32)
    g1n_ref[...] = dis * jnp.dot(v, w2p_ref[...],
                                 preferred_element_type=jnp.float32)


def _tcf_body(s2p_ref, dis_ref, u0_ref, b_ref, bg_ref, bb_ref, xprev_ref,
              eacc_ref, lw_ref, ow_ref, ob_ref, out_ref):
    dis = dis_ref[...]
    s2 = s2p_ref[...][0] + s2p_ref[...][1]
    cheb = u0_ref[...] - dis * s2[:, 0:_H] + b_ref[...][0]
    scale = 1.0 / jnp.sqrt(jnp.float32(1.0 + 1e-05))
    v = jax.nn.relu(cheb * scale * bg_ref[...][0] + bb_ref[...][0]) \
        + 0.7 * xprev_ref[...]
    w = jax.nn.softmax(lw_ref[...][0])
    emb = eacc_ref[...] + w[3] * v
    out_ref[...] = jnp.dot(emb, ow_ref[...].T,
                           preferred_element_type=jnp.float32) + ob_ref[...][0]


def _full(shape):
    return pl.BlockSpec(shape, lambda *_: tuple(0 for _ in shape))


_tca = pl.pallas_call(
    _tca_body,
    grid=(_N // _RB,),
    in_specs=[
        pl.BlockSpec((_RB, _D_IN), lambda i: (i, 0)),
        pl.BlockSpec((_D_IN, _H), lambda i: (0, 0)),
        pl.BlockSpec((_D_IN, _DQ), lambda i: (0, 0)),
        pl.BlockSpec((_D_IN, _DQ), lambda i: (0, 0)),
        pl.BlockSpec((2, _RB, _DD), lambda i: (0, i, 0)),
    ],
    out_specs=[
        pl.BlockSpec((_RB, _H), lambda i: (i, 0)),
        pl.BlockSpec((_RB, _DQ), lambda i: (i, 0)),
        pl.BlockSpec((_RB, _DQ), lambda i: (i, 0)),
        pl.BlockSpec((_RB, 1), lambda i: (i, 0)),
    ],
    out_shape=[
        jax.ShapeDtypeStruct((_N, _H), jnp.float32),
        jax.ShapeDtypeStruct((_N, _DQ), jnp.float32),
        jax.ShapeDtypeStruct((_N, _DQ), jnp.float32),
        jax.ShapeDtypeStruct((_N, 1), jnp.float32),
    ],
)

def _rows(w):
    return pl.BlockSpec((_RB, w), lambda i: (i, 0))


def _part(w):
    return pl.BlockSpec((2, _RB, w), lambda i: (0, i, 0))


_tcb = pl.pallas_call(
    _tcb_body,
    grid=(_N // _RB,),
    in_specs=[_part(_DQ), _rows(1), _rows(_DQ)],
    out_specs=_rows(_DQ),
    out_shape=jax.ShapeDtypeStruct((_N, _DQ), jnp.float32),
)


def _make_tcc(has_resid, lw_idx):
    return pl.pallas_call(
        functools.partial(_tcc_body, has_resid, lw_idx),
        grid=(_N // _RB,),
        in_specs=[
            _part(_DQ), _rows(1), _rows(_H),
            _full((1, _H)), _full((1, _H)), _full((1, _H)),
            _rows(_H), _rows(_H), _full((1, 4)),
            _full((_H, _H)), _full((_H, _DQ)), _full((_H, _DQ)),
        ],
        out_specs=[
            _rows(_H), _rows(_H), _rows(_H), _rows(_DQ), _rows(_DQ),
        ],
        out_shape=[
            jax.ShapeDtypeStruct((_N, _H), jnp.float32),
            jax.ShapeDtypeStruct((_N, _H), jnp.float32),
            jax.ShapeDtypeStruct((_N, _H), jnp.float32),
            jax.ShapeDtypeStruct((_N, _DQ), jnp.float32),
            jax.ShapeDtypeStruct((_N, _DQ), jnp.float32),
        ],
    )


_tcf = pl.pallas_call(
    _tcf_body,
    grid=(_N // _RB,),
    in_specs=[
        _part(_DQ), _rows(1), _rows(_H),
        _full((1, _H)), _full((1, _H)), _full((1, _H)),
        _rows(_H), _rows(_H), _full((1, 4)),
        _full((2, _H)), _full((1, 2)),
    ],
    out_specs=_rows(2),
    out_shape=jax.ShapeDtypeStruct((_N, 2), jnp.float32),
)


def _pad_w(w):
    return jnp.concatenate(
        [w, jnp.zeros((w.shape[0], _DQ - w.shape[1]), jnp.float32)], axis=1)


def kernel(features, edges, edge_weight, c0w0, c0w1, c0w2, c0b, c1w0, c1w1,
           c1w2, c1b, c2w0, c2w1, c2w2, c2b, c3w0, c3w1, c3w2, c3b, bn0g,
           bn0b, bn1g, bn1b, bn2g, bn2b, bn3g, bn3b, out_w, out_b, layer_w):
    src = edges[0]
    dst = edges[1]
    npad = _EPAD - _E
    nrow = _EPAD // _EB
    src_q = jnp.concatenate([src, jnp.zeros((npad,), jnp.int32)])
    src_q = src_q.reshape(nrow, _EB)
    dst_q = jnp.concatenate([dst, jnp.full((npad,), _N, jnp.int32)])
    dst_q = dst_q.reshape(nrow, _EB)
    src_d = jnp.concatenate([src, jnp.full((npad,), _N, jnp.int32)])
    src_d = src_d.reshape(nrow, _EB)

    ws = [(c0w0, c0w1, c0w2), (c1w0, c1w1, c1w2), (c2w0, c2w1, c2w2),
          (c3w0, c3w1, c3w2)]
    w0c = [w0 - w2 for (w0, w1, w2) in ws]
    w1p = [_pad_w(w1) for (w0, w1, w2) in ws]
    w2p = [_pad_w(w2) for (w0, w1, w2) in ws]
    bs = [c0b, c1b, c2b, c3b]
    bgs = [bn0g, bn1g, bn2g, bn3g]
    bbs = [bn0b, bn1b, bn2b, bn3b]

    def row(v):
        return v.reshape(1, -1)

    zeros_ = jnp.zeros((_RPT, _DQ), jnp.float32)
    ones_ = jnp.ones((_EB, _DD), jnp.float32)

    degp = _deg_sc(src_d, zeros_, ones_)
    u0, u1, g1, dis = _tca(features, w0c[0], w1p[0], w2p[0], degp)

    x = jnp.zeros((_N, _H), jnp.float32)
    eacc = jnp.zeros((_N, _H), jnp.float32)
    for i in range(4):
        s1p = _q_sc(g1, src_q, dst_q, zeros_)
        g2 = _tcb(s1p, dis, u1)
        s2p = _q_sc(g2, src_q, dst_q, zeros_)
        if i < 3:
            x, eacc, u0, u1, g1 = _make_tcc(i > 0, i)(
                s2p, dis, u0, row(bs[i]), row(bgs[i]), row(bbs[i]), x, eacc,
                row(layer_w), w0c[i + 1], w1p[i + 1], w2p[i + 1])
        else:
            out = _tcf(s2p, dis, u0, row(bs[i]), row(bgs[i]), row(bbs[i]), x,
                       eacc, row(layer_w), out_w, row(out_b))
    return out
